# baseline (speedup 1.0000x reference)
import functools

import jax
import jax.numpy as jnp
from jax import lax
from jax.experimental import pallas as pl
from jax.experimental.pallas import tpu as pltpu
from jax.experimental.pallas import tpu_sc as plsc

_N = 16384
_C = 1000
_D = 128

_NW = 32
_RPW = _N // _NW
_G = 16
_NGRP = _RPW // _G
_NBUF = 4
_NACC = 4
_NOBUF = 2
_NCHUNK = (_C + 15) // 16


def _lane_permute(v, perm):
    dnums = lax.GatherDimensionNumbers(
        offset_dims=(), collapsed_slice_dims=(0,), start_index_map=(0,))
    return lax.gather(
        v, perm[:, None], dnums, slice_sizes=(1,),
        mode=lax.GatherScatterMode.PROMISE_IN_BOUNDS)


def _merge(m1, i1, m2, i2):
    better = (m2 > m1) | ((m2 == m1) & (i2 < i1))
    return jnp.where(better, m2, m1), jnp.where(better, i2, i1)


def _row_argmax(row_ref, r, lane):
    vmax = [jnp.full((16,), -jnp.inf, dtype=jnp.float32)
            for _ in range(_NACC)]
    vidx = [jnp.zeros((16,), dtype=jnp.int32) for _ in range(_NACC)]
    for j in range(_NCHUNK):
        a = j % _NACC
        off = min(j * 16, _C - 16)
        v = row_ref[r, pl.ds(off, 16)]
        m = v > vmax[a]
        vmax[a] = jnp.where(m, v, vmax[a])
        vidx[a] = jnp.where(m, lane + off, vidx[a])
    while len(vmax) > 1:
        nm, ni = [], []
        for k in range(0, len(vmax), 2):
            a, b = _merge(vmax[k], vidx[k], vmax[k + 1], vidx[k + 1])
            nm.append(a)
            ni.append(b)
        vmax, vidx = nm, ni
    vm, vi = vmax[0], vidx[0]
    for s in (8, 4, 2, 1):
        perm = lane ^ s
        vm, vi = _merge(vm, vi, _lane_permute(vm, perm),
                        _lane_permute(vi, perm))
    return vi


@functools.cache
def _make_sc_kernel():
    mesh = plsc.VectorSubcoreMesh(core_axis_name="c", subcore_axis_name="s")

    @pl.kernel(
        mesh=mesh,
        out_type=jax.ShapeDtypeStruct((_N, _D), jnp.float32),
        scratch_types=[
            pltpu.VMEM((_NBUF, _G, _C), jnp.float32),
            pltpu.VMEM((_NOBUF, _G, _D), jnp.float32),
        ] + [pltpu.SemaphoreType.DMA] * (_NBUF + 1 + _NOBUF),
    )
    def enc(attr_hbm, table_hbm, out_hbm, inbuf, gbuf, *sems):
        insems = sems[:_NBUF]
        gsem = sems[_NBUF]
        osems = sems[_NBUF + 1:]
        w = lax.axis_index("s") * 2 + lax.axis_index("c")
        base = w * _RPW
        lane = lax.iota(jnp.int32, 16)

        def in_slice(g):
            return attr_hbm.at[pl.ds(base + g * _G, _G)]

        def out_slice(g):
            return out_hbm.at[pl.ds(base + g * _G, _G)]

        for b in range(_NBUF):
            pltpu.async_copy(in_slice(b), inbuf.at[b], insems[b])

        def group_body(i, _):
            for b in range(_NBUF):
                g = i * _NBUF + b
                ib = inbuf.at[b]
                ob = gbuf.at[b % _NOBUF]
                osem = osems[b % _NOBUF]
                pltpu.make_async_copy(in_slice(g), ib, insems[b]).wait()

                def row_body(r, idxvec):
                    rowidx = _row_argmax(ib, r, lane)
                    return jnp.where(lane == r, rowidx, idxvec)

                idxvec = lane

                @pl.when(g + _NBUF < _NGRP)
                def _():
                    pltpu.async_copy(in_slice(g + _NBUF), ib, insems[b])

                @pl.when(g >= _NOBUF)
                def _():
                    pltpu.make_async_copy(ob, out_slice(g), osem).wait()

                pltpu.async_copy(table_hbm.at[idxvec], ob, gsem).wait()
                pltpu.async_copy(ob, out_slice(g), osem)
            return ()

        lax.fori_loop(0, _NGRP // _NBUF, group_body, ())

        for g in range(_NGRP - _NOBUF, _NGRP):
            pltpu.make_async_copy(
                gbuf.at[g % _NOBUF], out_slice(g), osems[g % _NOBUF]).wait()

    return enc


def kernel(frag_attr, embedding_weight):
    return _make_sc_kernel()(frag_attr, embedding_weight)


# speedup vs baseline: 1.2699x; 1.2699x over previous
import functools

import jax
import jax.numpy as jnp
from jax import lax
from jax.experimental import pallas as pl
from jax.experimental.pallas import tpu as pltpu
from jax.experimental.pallas import tpu_sc as plsc

_N = 16384
_C = 1000
_D = 128

_NW = 32
_RPW = _N // _NW
_G = 16
_NGRP = _RPW // _G
_NBUF = 4
_NACC = 4
_SG = 128
_GPS = _SG // _G
_NSG = _RPW // _SG
_NCHUNK = (_C + 15) // 16


def _lane_permute(v, perm):
    dnums = lax.GatherDimensionNumbers(
        offset_dims=(), collapsed_slice_dims=(0,), start_index_map=(0,))
    return lax.gather(
        v, perm[:, None], dnums, slice_sizes=(1,),
        mode=lax.GatherScatterMode.PROMISE_IN_BOUNDS)


def _merge(m1, i1, m2, i2):
    better = (m2 > m1) | ((m2 == m1) & (i2 < i1))
    return jnp.where(better, m2, m1), jnp.where(better, i2, i1)


def _row_argmax(row_ref, r, lane):
    vmax = [jnp.full((16,), -jnp.inf, dtype=jnp.float32)
            for _ in range(_NACC)]
    vidx = [jnp.zeros((16,), dtype=jnp.int32) for _ in range(_NACC)]
    for j in range(_NCHUNK):
        a = j % _NACC
        off = min(j * 16, _C - 16)
        v = row_ref[r, pl.ds(off, 16)]
        m = v > vmax[a]
        vmax[a] = jnp.where(m, v, vmax[a])
        vidx[a] = jnp.where(m, lane + off, vidx[a])
    while len(vmax) > 1:
        nm, ni = [], []
        for k in range(0, len(vmax), 2):
            a, b = _merge(vmax[k], vidx[k], vmax[k + 1], vidx[k + 1])
            nm.append(a)
            ni.append(b)
        vmax, vidx = nm, ni
    vm, vi = vmax[0], vidx[0]
    for s in (8, 4, 2, 1):
        perm = lane ^ s
        vm, vi = _merge(vm, vi, _lane_permute(vm, perm),
                        _lane_permute(vi, perm))
    return vi


@functools.cache
def _make_sc_kernel():
    mesh = plsc.VectorSubcoreMesh(core_axis_name="c", subcore_axis_name="s")

    @pl.kernel(
        mesh=mesh,
        out_type=jax.ShapeDtypeStruct((_N, _D), jnp.float32),
        scratch_types=[
            pltpu.VMEM((_NBUF, _G, _C), jnp.float32),
            pltpu.VMEM((2, _SG), jnp.int32),
            pltpu.VMEM((2, _SG, _D), jnp.float32),
            pltpu.SemaphoreType.DMA((_NBUF,)),
            pltpu.SemaphoreType.DMA,
            pltpu.SemaphoreType.DMA((2,)),
        ],
    )
    def enc(attr_hbm, table_hbm, out_hbm, inbuf, idxbuf, ostage,
            insems, gsem, osems):
        w = lax.axis_index("s") * 2 + lax.axis_index("c")
        base = w * _RPW
        lane = lax.iota(jnp.int32, 16)

        def in_slice(g):
            return attr_hbm.at[pl.ds(base + g * _G, _G)]

        for b in range(_NBUF):
            pltpu.async_copy(in_slice(b), inbuf.at[b], insems.at[b])

        def group_body(g, _):
            b = lax.rem(g, _NBUF)
            s = lax.div(g, _GPS)
            k = lax.rem(g, _GPS)
            sl = lax.rem(s, 2)
            ib = inbuf.at[b]
            pltpu.make_async_copy(in_slice(g), ib, insems.at[b]).wait()

            def row_body(r, idxvec):
                rowidx = _row_argmax(ib, r, lane)
                return jnp.where(lane == r, rowidx, idxvec)

            idxvec = lax.fori_loop(
                0, _G, row_body, jnp.zeros((16,), jnp.int32))
            idxbuf[sl, pl.ds(k * _G, _G)] = idxvec

            @pl.when(g + _NBUF < _NGRP)
            def _():
                pltpu.async_copy(in_slice(g + _NBUF), ib, insems.at[b])

            @pl.when(k == _GPS - 1)
            def _():
                ost = ostage.at[sl]

                @pl.when(s >= 2)
                def _():
                    pltpu.make_async_copy(
                        ost, out_hbm.at[pl.ds(base, _SG)],
                        osems.at[sl]).wait()

                pltpu.async_copy(
                    table_hbm.at[idxbuf.at[sl]], ost, gsem).wait()
                pltpu.async_copy(
                    ost, out_hbm.at[pl.ds(base + s * _SG, _SG)],
                    osems.at[sl])
            return ()

        lax.fori_loop(0, _NGRP, group_body, ())

        for s in range(_NSG - 2, _NSG):
            pltpu.make_async_copy(
                ostage.at[s % 2],
                out_hbm.at[pl.ds(base + s * _SG, _SG)],
                osems.at[s % 2]).wait()

    return enc


def kernel(frag_attr, embedding_weight):
    return _make_sc_kernel()(frag_attr, embedding_weight)


# speedup vs baseline: 3.2106x; 2.5282x over previous
import functools

import jax
import jax.numpy as jnp
from jax import lax
from jax.experimental import pallas as pl
from jax.experimental.pallas import tpu as pltpu
from jax.experimental.pallas import tpu_sc as plsc

_N = 16384
_C = 1000
_D = 128

_COLS_PER_BLOCK = 2048

_NW = 32
_BPW = _N // _NW
_CHUNK = 128
_NCHUNK = _BPW // _CHUNK


def _argmax_block(xt_ref, idx_ref):
    x = xt_ref[...]
    maxv = jnp.max(x, axis=0, keepdims=True)
    cls = lax.broadcasted_iota(jnp.int32, x.shape, 0)
    cand = jnp.where(x == maxv, cls, jnp.int32(_C))
    idx_ref[...] = jnp.min(cand, axis=0)


@functools.cache
def _make_sc_gather():
    mesh = plsc.VectorSubcoreMesh(core_axis_name="c", subcore_axis_name="s")

    @pl.kernel(
        mesh=mesh,
        out_type=jax.ShapeDtypeStruct((_N, _D), jnp.float32),
        scratch_types=[
            pltpu.VMEM((_NCHUNK, _CHUNK), jnp.int32),
            pltpu.VMEM((_BPW, _D), jnp.float32),
            pltpu.SemaphoreType.DMA,
        ],
    )
    def gather(idx_hbm, table_hbm, out_hbm, idx_v, rows_v, sem):
        w = lax.axis_index("s") * 2 + lax.axis_index("c")
        base = w * _BPW
        pltpu.sync_copy(idx_hbm.at[w], idx_v)
        copies = [
            pltpu.async_copy(
                table_hbm.at[idx_v.at[j]],
                rows_v.at[pl.ds(j * _CHUNK, _CHUNK)],
                sem,
            )
            for j in range(_NCHUNK)
        ]
        for cp in copies:
            cp.wait()
        pltpu.sync_copy(rows_v, out_hbm.at[pl.ds(base, _BPW)])

    return gather


def kernel(frag_attr, embedding_weight):
    xt = frag_attr.T
    idx = pl.pallas_call(
        _argmax_block,
        grid=(_N // _COLS_PER_BLOCK,),
        in_specs=[pl.BlockSpec((_C, _COLS_PER_BLOCK), lambda i: (0, i))],
        out_specs=pl.BlockSpec((_COLS_PER_BLOCK,), lambda i: (i,)),
        out_shape=jax.ShapeDtypeStruct((_N,), jnp.int32),
    )(xt)
    idx3 = idx.reshape(_NW, _NCHUNK, _CHUNK)
    return _make_sc_gather()(idx3, embedding_weight)
